# Initial kernel scaffold; baseline (speedup 1.0000x reference)
import jax
import jax.numpy as jnp
from jax.experimental import pallas as pl


def kernel(y, target, x, edge_index, W_msg, b_msg):
    raise NotImplementedError("write your pallas kernel here")


# speedup vs baseline: 8.9303x; 8.9303x over previous
import functools

import jax
import jax.numpy as jnp
from jax import lax
from jax.experimental import pallas as pl
from jax.experimental.pallas import tpu as pltpu
from jax.experimental.pallas import tpu_sc as plsc

REG_WEIGHT_ = 0.01
NC = 2
NS = 16
NW = NC * NS
CW = 128


def _tables_body(n_nodes, x_ref, w_ref, b_ref, p_ref, q_ref):
    pq = jnp.dot(x_ref[...], w_ref[...], preferred_element_type=jnp.float32)
    m = pq.shape[1] // 2
    rows = lax.broadcasted_iota(jnp.int32, (pq.shape[0], m), 0)
    valid = rows < n_nodes
    p_ref[...] = jnp.where(valid, pq[:, :m] + b_ref[...], 0.0)
    q_ref[...] = jnp.where(valid, pq[:, m:], 0.0)


def _combine_body(n_nodes, n_edges, y_ref, t_ref, part_ref, o_ref):
    base = jnp.sum(jnp.abs(y_ref[...] - t_ref[...]))
    l1 = jnp.sum(part_ref[...])
    total = base / n_nodes + REG_WEIGHT_ * (l1 / n_edges)
    o_ref[...] = jnp.reshape(total, (1, 1))


def _make_edge_l1(nchunk, msg_dim):
    mesh = plsc.VectorSubcoreMesh(core_axis_name="c", subcore_axis_name="s")

    @functools.partial(
        pl.kernel,
        mesh=mesh,
        out_type=jax.ShapeDtypeStruct((NW, msg_dim), jnp.float32),
        compiler_params=pltpu.CompilerParams(use_tc_tiling_on_sc=False),
        scratch_types=[
            pltpu.VMEM((nchunk, CW), jnp.int32),
            pltpu.VMEM((nchunk, CW), jnp.int32),
            pltpu.VMEM((CW, msg_dim), jnp.float32),
            pltpu.VMEM((CW, msg_dim), jnp.float32),
            pltpu.VMEM((msg_dim,), jnp.float32),
            pltpu.SemaphoreType.DMA,
            pltpu.SemaphoreType.DMA,
        ],
    )
    def edge_l1(p_hbm, q_hbm, src_hbm, dst_hbm, out_hbm,
                sidx, didx, pbuf, qbuf, accv, sem_p, sem_q):
        wid = lax.axis_index("s") * NC + lax.axis_index("c")
        base_row = wid * nchunk
        pltpu.sync_copy(src_hbm.at[pl.ds(base_row, nchunk)], sidx)
        pltpu.sync_copy(dst_hbm.at[pl.ds(base_row, nchunk)], didx)

        def chunk_body(c, acc):
            cp = pltpu.async_copy(p_hbm.at[sidx.at[c]], pbuf, sem_p)
            cq = pltpu.async_copy(q_hbm.at[didx.at[c]], qbuf, sem_q)
            cp.wait()
            cq.wait()

            def lane_body(i, a):
                return a + jnp.abs(pbuf[i] + qbuf[i])

            return lax.fori_loop(0, CW, lane_body, acc)

        acc = lax.fori_loop(0, nchunk, chunk_body,
                            jnp.zeros((msg_dim,), jnp.float32))
        accv[...] = acc
        pltpu.sync_copy(accv, out_hbm.at[wid])

    return edge_l1


def kernel(y, target, x, edge_index, W_msg, b_msg):
    n_nodes, d_feat = x.shape
    n_edges = edge_index.shape[1]
    msg_dim = W_msg.shape[1]

    nchunk = -(-n_edges // (NW * CW))
    nchunk = -(-nchunk // 8) * 8
    e_pad = NW * nchunk * CW
    n_pad = -(-(n_nodes + 1) // 8) * 8

    src = edge_index[0].astype(jnp.int32)
    dst = edge_index[1].astype(jnp.int32)
    fill = jnp.full((e_pad,), n_nodes, jnp.int32)
    src_p = fill.at[:n_edges].set(src).reshape(NW * nchunk, CW)
    dst_p = fill.at[:n_edges].set(dst).reshape(NW * nchunk, CW)

    x_p = jnp.zeros((n_pad, d_feat), x.dtype).at[:n_nodes].set(x)
    w2 = jnp.concatenate([W_msg[:d_feat], W_msg[d_feat:]], axis=1)
    b2 = b_msg.reshape(1, msg_dim)

    tables = pl.pallas_call(
        functools.partial(_tables_body, n_nodes),
        out_shape=(jax.ShapeDtypeStruct((n_pad, msg_dim), jnp.float32),
                   jax.ShapeDtypeStruct((n_pad, msg_dim), jnp.float32)),
    )
    p_tab, q_tab = tables(x_p, w2, b2)

    partials = _make_edge_l1(nchunk, msg_dim)(p_tab, q_tab, src_p, dst_p)

    y2 = y.reshape(80, -1)
    t2 = target.reshape(80, -1)
    combine = pl.pallas_call(
        functools.partial(_combine_body, n_nodes, n_edges),
        out_shape=jax.ShapeDtypeStruct((1, 1), jnp.float32),
    )
    return combine(y2, t2, partials)[0, 0]


# speedup vs baseline: 13.3975x; 1.5002x over previous
import functools

import jax
import jax.numpy as jnp
from jax import lax
from jax.experimental import pallas as pl
from jax.experimental.pallas import tpu as pltpu
from jax.experimental.pallas import tpu_sc as plsc

REG_WEIGHT_ = 0.01
NC = 2
NS = 16
NW = NC * NS
CW = 128


def _tables_body(n_nodes, x_ref, w_ref, b_ref, p_ref, q_ref):
    pq = jnp.dot(x_ref[...], w_ref[...], preferred_element_type=jnp.float32)
    m = pq.shape[1] // 2
    rows = lax.broadcasted_iota(jnp.int32, (pq.shape[0], m), 0)
    valid = rows < n_nodes
    p_ref[...] = jnp.where(valid, pq[:, :m] + b_ref[...], 0.0)
    q_ref[...] = jnp.where(valid, pq[:, m:], 0.0)


def _combine_body(n_nodes, n_edges, y_ref, t_ref, part_ref, o_ref):
    base = jnp.sum(jnp.abs(y_ref[...] - t_ref[...]))
    l1 = jnp.sum(part_ref[...])
    total = base / n_nodes + REG_WEIGHT_ * (l1 / n_edges)
    o_ref[...] = jnp.reshape(total, (1, 1))


def _make_edge_l1(nchunk, msg_dim):
    mesh = plsc.VectorSubcoreMesh(core_axis_name="c", subcore_axis_name="s")

    @functools.partial(
        pl.kernel,
        mesh=mesh,
        out_type=jax.ShapeDtypeStruct((NW, msg_dim), jnp.float32),
        compiler_params=pltpu.CompilerParams(use_tc_tiling_on_sc=False),
        scratch_types=[
            pltpu.VMEM((nchunk, CW), jnp.int32),
            pltpu.VMEM((nchunk, CW), jnp.int32),
            pltpu.VMEM((CW, msg_dim), jnp.float32),
            pltpu.VMEM((CW, msg_dim), jnp.float32),
            pltpu.VMEM((CW, msg_dim), jnp.float32),
            pltpu.VMEM((CW, msg_dim), jnp.float32),
            pltpu.VMEM((msg_dim,), jnp.float32),
            pltpu.SemaphoreType.DMA,
            pltpu.SemaphoreType.DMA,
            pltpu.SemaphoreType.DMA,
            pltpu.SemaphoreType.DMA,
        ],
    )
    def edge_l1(p_hbm, q_hbm, src_hbm, dst_hbm, out_hbm,
                sidx, didx, pbuf0, qbuf0, pbuf1, qbuf1, accv,
                sem_p0, sem_q0, sem_p1, sem_q1):
        wid = lax.axis_index("s") * NC + lax.axis_index("c")
        base_row = wid * nchunk
        pltpu.sync_copy(src_hbm.at[pl.ds(base_row, nchunk)], sidx)
        pltpu.sync_copy(dst_hbm.at[pl.ds(base_row, nchunk)], didx)

        def issue(c, pb, qb, sp, sq):
            pltpu.async_copy(p_hbm.at[sidx.at[c]], pb, sp)
            pltpu.async_copy(q_hbm.at[didx.at[c]], qb, sq)

        def drain(c, pb, qb, sp, sq):
            pltpu.make_async_copy(p_hbm.at[sidx.at[c]], pb, sp).wait()
            pltpu.make_async_copy(q_hbm.at[didx.at[c]], qb, sq).wait()

        def accum(pb, qb, acc):
            def lane_body(i, carry):
                a0, a1 = carry
                j = i * 2
                a0 = a0 + jnp.abs(pb[j] + qb[j])
                a1 = a1 + jnp.abs(pb[j + 1] + qb[j + 1])
                return a0, a1

            a0, a1 = lax.fori_loop(0, CW // 2, lane_body,
                                   (acc, jnp.zeros((msg_dim,), jnp.float32)),
                                   unroll=4)
            return a0 + a1

        issue(0, pbuf0, qbuf0, sem_p0, sem_q0)

        def pair_body(k, acc):
            c = k * 2
            issue(c + 1, pbuf1, qbuf1, sem_p1, sem_q1)
            drain(c, pbuf0, qbuf0, sem_p0, sem_q0)
            acc = accum(pbuf0, qbuf0, acc)

            @pl.when(c + 2 < nchunk)
            def _():
                issue(c + 2, pbuf0, qbuf0, sem_p0, sem_q0)

            drain(c + 1, pbuf1, qbuf1, sem_p1, sem_q1)
            return accum(pbuf1, qbuf1, acc)

        acc = lax.fori_loop(0, nchunk // 2, pair_body,
                            jnp.zeros((msg_dim,), jnp.float32))
        accv[...] = acc
        pltpu.sync_copy(accv, out_hbm.at[wid])

    return edge_l1


def kernel(y, target, x, edge_index, W_msg, b_msg):
    n_nodes, d_feat = x.shape
    n_edges = edge_index.shape[1]
    msg_dim = W_msg.shape[1]

    nchunk = -(-n_edges // (NW * CW))
    nchunk = -(-nchunk // 8) * 8
    e_pad = NW * nchunk * CW
    n_pad = -(-(n_nodes + 1) // 8) * 8

    src = edge_index[0].astype(jnp.int32)
    dst = edge_index[1].astype(jnp.int32)
    fill = jnp.full((e_pad,), n_nodes, jnp.int32)
    src_p = fill.at[:n_edges].set(src).reshape(NW * nchunk, CW)
    dst_p = fill.at[:n_edges].set(dst).reshape(NW * nchunk, CW)

    x_p = jnp.zeros((n_pad, d_feat), x.dtype).at[:n_nodes].set(x)
    w2 = jnp.concatenate([W_msg[:d_feat], W_msg[d_feat:]], axis=1)
    b2 = b_msg.reshape(1, msg_dim)

    tables = pl.pallas_call(
        functools.partial(_tables_body, n_nodes),
        out_shape=(jax.ShapeDtypeStruct((n_pad, msg_dim), jnp.float32),
                   jax.ShapeDtypeStruct((n_pad, msg_dim), jnp.float32)),
    )
    p_tab, q_tab = tables(x_p, w2, b2)

    partials = _make_edge_l1(nchunk, msg_dim)(p_tab, q_tab, src_p, dst_p)

    y2 = y.reshape(80, -1)
    t2 = target.reshape(80, -1)
    combine = pl.pallas_call(
        functools.partial(_combine_body, n_nodes, n_edges),
        out_shape=jax.ShapeDtypeStruct((1, 1), jnp.float32),
    )
    return combine(y2, t2, partials)[0, 0]
